# baseline (speedup 1.0000x reference)
import jax
import jax.numpy as jnp
from jax.experimental import pallas as pl

_NUM_EXPERTS = 64
_TOP_K = 8
_TB = 1024


def _route_tile(logits):
    tb = logits.shape[1]
    fiota = jax.lax.broadcasted_iota(jnp.int32, (_NUM_EXPERTS, tb), 0).astype(
        jnp.float32
    )
    work = logits
    idx_rows = []
    top_val = None
    neg_inf = jnp.float32(-jnp.inf)
    for k in range(_TOP_K):
        m = jnp.max(work, axis=0, keepdims=True)
        if k == 0:
            top_val = m
        idx = jnp.min(
            jnp.where(work == m, fiota, jnp.float32(_NUM_EXPERTS)),
            axis=0,
            keepdims=True,
        )
        work = jnp.where(fiota == idx, neg_inf, work)
        idx_rows.append(idx)

    e = jnp.where(work == neg_inf, jnp.exp(logits - top_val), 0.0)
    denom = jnp.sum(e, axis=0, keepdims=True)
    idxs = jnp.concatenate(idx_rows, axis=0)
    return (e / denom).T, idxs.astype(jnp.int32)


def _router_block(xe_ref, xo_ref, w_ref, b_ref, out_ref, idx_ref):
    w = w_ref[...]
    for s, x_ref in ((0, xe_ref), (1, xo_ref)):
        logits = jax.lax.dot_general(
            w,
            x_ref[:, 0, 0, :],
            (((1,), (1,)), ((), ())),
            preferred_element_type=jnp.float32,
        )
        logits = logits + b_ref[...]
        probs, idxs = _route_tile(logits)
        out_ref[:, pl.ds(s * _NUM_EXPERTS, _NUM_EXPERTS)] = probs
        idx_ref[s, :, :] = idxs


@jax.jit
def kernel(mh_output, W, b):
    B, S, E = mh_output.shape
    n_tok = B * S
    hb = _TB // 2
    xv = mh_output.reshape(n_tok // 2, 2, 1, E)
    grid = (n_tok // _TB,)
    router, idx = pl.pallas_call(
        _router_block,
        grid=grid,
        in_specs=[
            pl.BlockSpec((hb, 1, 1, E), lambda i: (i, 0, 0, 0)),
            pl.BlockSpec((hb, 1, 1, E), lambda i: (i, 1, 0, 0)),
            pl.BlockSpec((_NUM_EXPERTS, E), lambda i: (0, 0)),
            pl.BlockSpec((_NUM_EXPERTS, 1), lambda i: (0, 0)),
        ],
        out_specs=[
            pl.BlockSpec((hb, 2 * _NUM_EXPERTS), lambda i: (i, 0)),
            pl.BlockSpec((2, _TOP_K, hb), lambda i: (0, 0, i)),
        ],
        out_shape=[
            jax.ShapeDtypeStruct((n_tok // 2, 2 * _NUM_EXPERTS), jnp.float32),
            jax.ShapeDtypeStruct((2, _TOP_K, n_tok // 2), jnp.int32),
        ],
    )(xv, xv, W, b.reshape(_NUM_EXPERTS, 1))
    router = router.reshape(B, S, _NUM_EXPERTS)
    idx = jnp.transpose(idx, (2, 0, 1)).reshape(B, S, _TOP_K)
    return router, idx


# speedup vs baseline: 11.2347x; 11.2347x over previous
import jax
import jax.numpy as jnp
from jax.experimental import pallas as pl

_NUM_EXPERTS = 64
_TOP_K = 8
_TB = 1024
_SUB = 2


def _route_tile(logits):
    tb = logits.shape[1]
    fiota = jax.lax.broadcasted_iota(jnp.int32, (_NUM_EXPERTS, tb), 0).astype(
        jnp.float32
    )
    work = logits
    idx_rows = []
    top_val = None
    neg_inf = jnp.float32(-jnp.inf)
    for k in range(_TOP_K):
        m = jnp.max(work, axis=0, keepdims=True)
        if k == 0:
            top_val = m
        idx = jnp.min(
            jnp.where(work == m, fiota, jnp.float32(_NUM_EXPERTS)),
            axis=0,
            keepdims=True,
        )
        work = jnp.where(fiota == idx, neg_inf, work)
        idx_rows.append(idx)

    e = jnp.where(work == neg_inf, jnp.exp(logits - top_val), 0.0)
    denom = jnp.sum(e, axis=0, keepdims=True)
    idxs = jnp.concatenate(idx_rows, axis=0)
    return e / denom, idxs.astype(jnp.int32)


def _router_block(x_ref, w_ref, b_ref, out_ref, idx_ref):
    w = w_ref[...]
    st = _TB // _SUB
    for s in range(_SUB):
        logits = jax.lax.dot_general(
            w,
            x_ref[pl.ds(s * st, st), :],
            (((1,), (1,)), ((), ())),
            preferred_element_type=jnp.float32,
        )
        logits = logits + b_ref[...]
        probs, idxs = _route_tile(logits)
        out_ref[:, pl.ds(s * st, st)] = probs
        idx_ref[:, pl.ds(s * st, st)] = idxs


@jax.jit
def kernel(mh_output, W, b):
    B, S, E = mh_output.shape
    n_tok = B * S
    x = mh_output.reshape(n_tok, E)
    grid = (n_tok // _TB,)
    router, idx = pl.pallas_call(
        _router_block,
        grid=grid,
        in_specs=[
            pl.BlockSpec((_TB, E), lambda i: (i, 0)),
            pl.BlockSpec((_NUM_EXPERTS, E), lambda i: (0, 0)),
            pl.BlockSpec((_NUM_EXPERTS, 1), lambda i: (0, 0)),
        ],
        out_specs=[
            pl.BlockSpec((_NUM_EXPERTS, _TB), lambda i: (0, i)),
            pl.BlockSpec((_TOP_K, _TB), lambda i: (0, i)),
        ],
        out_shape=[
            jax.ShapeDtypeStruct((_NUM_EXPERTS, n_tok), jnp.float32),
            jax.ShapeDtypeStruct((_TOP_K, n_tok), jnp.int32),
        ],
    )(x, W, b.reshape(_NUM_EXPERTS, 1))
    return router.T.reshape(B, S, _NUM_EXPERTS), idx.T.reshape(B, S, _TOP_K)
